# baseline (speedup 1.0000x reference)
import functools

import jax
import jax.numpy as jnp
from jax import lax
from jax.experimental import pallas as pl
from jax.experimental.pallas import tpu as pltpu
from jax.experimental.pallas import tpu_sc as plsc

_NC = 2
_NS = 16
_G = 128


def _segment_sum_sc(h, srcm, dstm, zeros):
  N, D = h.shape
  NW, NCHUNK, CH = srcm.shape
  NBUF = 5
  NI = 2 * NBUF
  NPAD = zeros.shape[0]
  RPT = NPAD // _NS

  mesh = plsc.VectorSubcoreMesh(core_axis_name="c", subcore_axis_name="s")

  @functools.partial(
      pl.kernel,
      out_type=(jax.ShapeDtypeStruct((NPAD, D), jnp.float32),
                jax.ShapeDtypeStruct((NPAD, D), jnp.float32)),
      mesh=mesh,
      scratch_types=[
          pltpu.VMEM((NI, CH), jnp.int32),
          pltpu.VMEM((NI, CH), jnp.int32),
          pltpu.VMEM((NBUF, CH, D), jnp.float32),
          pltpu.VMEM_SHARED((NPAD, D), jnp.float32),
          [pltpu.SemaphoreType.DMA] * NBUF,
          [pltpu.SemaphoreType.DMA] * NI,
          [pltpu.SemaphoreType.DMA] * NI,
          pltpu.SemaphoreType.DMA,
      ],
  )
  def seg_kernel(h_hbm, src_hbm, dst_hbm, z_hbm, out0_hbm, out1_hbm,
                 src_v, dst_v, rows_v, acc_sh, gsems, isems, dsems, sem):
    cid = lax.axis_index("c")
    sid = lax.axis_index("s")
    wid = sid * _NC + cid
    stripe = pl.ds(sid * RPT, RPT)

    def idx_start(i, b):
      pltpu.async_copy(src_hbm.at[wid, i], src_v.at[b], isems[b])
      pltpu.async_copy(dst_hbm.at[wid, i], dst_v.at[b], dsems[b])

    def idx_wait(i, b):
      pltpu.make_async_copy(src_hbm.at[wid, i], src_v.at[b], isems[b]).wait()
      pltpu.make_async_copy(dst_hbm.at[wid, i], dst_v.at[b], dsems[b]).wait()

    def gather_start(i, rb, b):
      pltpu.async_copy(h_hbm.at[src_v.at[b]], rows_v.at[rb], gsems[rb])

    def gather_wait(i, rb, b):
      pltpu.make_async_copy(h_hbm.at[src_v.at[b]], rows_v.at[rb],
                            gsems[rb]).wait()

    for j in range(NI):
      idx_start(j, j)
    pltpu.sync_copy(z_hbm.at[stripe], acc_sh.at[stripe])
    plsc.subcore_barrier()
    for k in range(NBUF):
      idx_wait(k, k)
      gather_start(k, k, k)

    def body(g, carry):
      for bb in range(NI):
        i = g + bb
        rb = bb % NBUF
        gather_wait(i, rb, bb)
        pltpu.sync_copy(rows_v.at[rb], acc_sh.at[dst_v.at[bb]], add=True)

        @pl.when(i + NI < NCHUNK)
        def _():
          idx_start(i + NI, bb)

        @pl.when(i + NBUF < NCHUNK)
        def _():
          idx_wait(i + NBUF, (bb + NBUF) % NI)
          gather_start(i + NBUF, rb, (bb + NBUF) % NI)
      return carry

    lax.fori_loop(0, NCHUNK // NI, lambda j, c: body(j * NI, c), 0,
                  unroll=False)
    plsc.subcore_barrier()

    @pl.when(cid == 0)
    def _():
      pltpu.sync_copy(acc_sh.at[stripe], out0_hbm.at[stripe])

    @pl.when(cid == 1)
    def _():
      pltpu.sync_copy(acc_sh.at[stripe], out1_hbm.at[stripe])

  return seg_kernel(h, srcm, dstm, zeros)


_BR = 1000


def _gin_layer(a0, a1, h, W1, b1, g1, be1, W2, b2, g2, be2, relu_last,
               pool=None):
  N, D = h.shape
  Hd = W1.shape[1]
  NB = N // _BR
  inv_n = 1.0 / N
  if pool is not None:
    batchb, Wc, bc = pool
    C = Wc.shape[1]

  def body(*refs):
    if pool is not None:
      (a0_ref, a1_ref, h_ref, w1_ref, b1_ref, g1_ref, be1_ref,
       w2_ref, b2_ref, g2_ref, be2_ref, bb_ref, wc_ref, bc_ref,
       hn_ref, out_ref, z_s, z2_s, s1_s, s2_s, t1_s, t2_s,
       ps_s, pc_s) = refs
    else:
      (a0_ref, a1_ref, h_ref, w1_ref, b1_ref, g1_ref, be1_ref,
       w2_ref, b2_ref, g2_ref, be2_ref,
       hn_ref, z_s, z2_s, s1_s, s2_s, t1_s, t2_s) = refs
    i = pl.program_id(0)
    phase = i // NB
    r = i - phase * NB
    row = pl.ds(r * _BR, _BR)

    @pl.when(i == 0)
    def _():
      s1_s[...] = jnp.zeros_like(s1_s)
      s2_s[...] = jnp.zeros_like(s2_s)
      t1_s[...] = jnp.zeros_like(t1_s)
      t2_s[...] = jnp.zeros_like(t2_s)
      if pool is not None:
        ps_s[...] = jnp.zeros_like(ps_s)
        pc_s[...] = jnp.zeros_like(pc_s)

    @pl.when(phase == 0)
    def _():
      xin = a0_ref[...] + a1_ref[...] + h_ref[...]
      z = jnp.dot(xin, w1_ref[...],
                  preferred_element_type=jnp.float32) + b1_ref[...]
      z_s[row, :] = z
      s1_s[...] += jnp.broadcast_to(
          jnp.sum(z, axis=0, keepdims=True), s1_s.shape)
      s2_s[...] += jnp.broadcast_to(
          jnp.sum(z * z, axis=0, keepdims=True), s2_s.shape)

    @pl.when(phase == 1)
    def _():
      m = s1_s[0:1, :] * inv_n
      v = s2_s[0:1, :] * inv_n - m * m
      scale = lax.rsqrt(v + 1e-5) * g1_ref[...]
      zr = jnp.maximum((z_s[row, :] - m) * scale + be1_ref[...], 0.0)
      z2 = jnp.dot(zr, w2_ref[...],
                   preferred_element_type=jnp.float32) + b2_ref[...]
      z2_s[row, :] = z2
      t1_s[...] += jnp.broadcast_to(
          jnp.sum(z2, axis=0, keepdims=True), t1_s.shape)
      t2_s[...] += jnp.broadcast_to(
          jnp.sum(z2 * z2, axis=0, keepdims=True), t2_s.shape)

    @pl.when(phase == 2)
    def _():
      m = t1_s[0:1, :] * inv_n
      v = t2_s[0:1, :] * inv_n - m * m
      scale = lax.rsqrt(v + 1e-5) * g2_ref[...]
      hn = (z2_s[row, :] - m) * scale + be2_ref[...]
      if relu_last:
        hn = jnp.maximum(hn, 0.0)
      hn_ref[...] = hn
      if pool is not None:
        gid = lax.broadcasted_iota(jnp.int32, (1, _G), 1)
        onehot = (bb_ref[...] == gid).astype(jnp.float32)
        ps_s[...] += lax.dot_general(
            onehot, hn, (((0,), (0,)), ((), ())),
            preferred_element_type=jnp.float32,
            precision=lax.Precision.HIGHEST)
        pc_s[...] += lax.dot_general(
            onehot, jnp.ones_like(hn), (((0,), (0,)), ((), ())),
            preferred_element_type=jnp.float32,
            precision=lax.Precision.HIGHEST)

    if pool is not None:
      @pl.when(phase == 3)
      def _():
        hg = ps_s[...] / jnp.maximum(pc_s[...], 1.0)
        out_ref[...] = jnp.dot(
            hg, wc_ref[...], preferred_element_type=jnp.float32) + bc_ref[...]

  nsteps = 3 * NB + (1 if pool is not None else 0)
  blk = lambda i: (jnp.minimum(i, NB - 1), 0)
  cst = lambda i: (0, 0)
  out_blk = lambda i: (jnp.clip(i - 2 * NB, 0, NB - 1), 0)
  in_specs = [
      pl.BlockSpec((_BR, D), blk),
      pl.BlockSpec((_BR, D), blk),
      pl.BlockSpec((_BR, D), blk),
      pl.BlockSpec((D, Hd), cst),
      pl.BlockSpec((1, Hd), cst),
      pl.BlockSpec((1, Hd), cst),
      pl.BlockSpec((1, Hd), cst),
      pl.BlockSpec((Hd, D), cst),
      pl.BlockSpec((1, D), cst),
      pl.BlockSpec((1, D), cst),
      pl.BlockSpec((1, D), cst),
  ]
  scratch = [
      pltpu.VMEM((N, Hd), jnp.float32),
      pltpu.VMEM((N, D), jnp.float32),
      pltpu.VMEM((8, Hd), jnp.float32),
      pltpu.VMEM((8, Hd), jnp.float32),
      pltpu.VMEM((8, D), jnp.float32),
      pltpu.VMEM((8, D), jnp.float32),
  ]
  out_specs = [pl.BlockSpec((_BR, D), out_blk)]
  out_shape = [jax.ShapeDtypeStruct((N, D), jnp.float32)]
  args = [a0, a1, h, W1, b1, g1, be1, W2, b2, g2, be2]
  if pool is not None:
    in_specs += [
        pl.BlockSpec((_BR, _G), out_blk),
        pl.BlockSpec((D, C), cst),
        pl.BlockSpec((1, C), cst),
    ]
    args += [batchb, Wc, bc]
    out_specs.append(pl.BlockSpec((_G, C), cst))
    out_shape.append(jax.ShapeDtypeStruct((_G, C), jnp.float32))
    scratch += [
        pltpu.VMEM((_G, D), jnp.float32),
        pltpu.VMEM((_G, D), jnp.float32),
    ]

  res = pl.pallas_call(
      body,
      grid=(nsteps,),
      in_specs=in_specs,
      out_specs=out_specs,
      out_shape=out_shape,
      scratch_shapes=scratch,
  )(*args)
  return res


def kernel(x, edge_index, edge_attr, batch, W1, b1, g1, be1,
           W2, b2, g2, be2, Wc, bc):
  del edge_attr
  N, D = x.shape
  L = W1.shape[0]
  E = edge_index.shape[1]
  NW = _NC * _NS
  CH = 40
  srcm = edge_index[0].reshape(NW, (E // NW) // CH, CH)
  dstm = edge_index[1].reshape(NW, (E // NW) // CH, CH)
  npad = ((N + _NS * 8 - 1) // (_NS * 8)) * (_NS * 8)
  zeros = jnp.zeros((npad, D), jnp.float32)
  batchb = jnp.broadcast_to(batch[:, None], (N, _G))

  h = x
  for l in range(L):
    a0, a1 = _segment_sum_sc(h, srcm, dstm, zeros)
    last = l == L - 1
    res = _gin_layer(
        a0, a1, h, W1[l], b1[l].reshape(1, -1), g1[l].reshape(1, -1),
        be1[l].reshape(1, -1), W2[l], b2[l].reshape(1, -1),
        g2[l].reshape(1, -1), be2[l].reshape(1, -1), relu_last=not last,
        pool=(batchb, Wc, bc.reshape(1, -1)) if last else None)
    h = res[0]

  return res[1]


# speedup vs baseline: 1.0434x; 1.0434x over previous
import functools

import jax
import jax.numpy as jnp
from jax import lax
from jax.experimental import pallas as pl
from jax.experimental.pallas import tpu as pltpu
from jax.experimental.pallas import tpu_sc as plsc

_NC = 2
_NS = 16
_G = 128


def _segment_sum_sc(h, srcm, dstm, zeros):
  N, D = h.shape
  NW, NCHUNK, CH = srcm.shape
  NBUF = 5
  NI = 2 * NBUF
  NPAD = zeros.shape[0]
  RPT = NPAD // _NS

  mesh = plsc.VectorSubcoreMesh(core_axis_name="c", subcore_axis_name="s")

  @functools.partial(
      pl.kernel,
      out_type=(jax.ShapeDtypeStruct((NPAD, D), jnp.float32),
                jax.ShapeDtypeStruct((NPAD, D), jnp.float32)),
      mesh=mesh,
      scratch_types=[
          pltpu.VMEM((NI, CH), jnp.int32),
          pltpu.VMEM((NI, CH), jnp.int32),
          pltpu.VMEM((NBUF, CH, D), jnp.float32),
          pltpu.VMEM_SHARED((NPAD, D), jnp.float32),
          [pltpu.SemaphoreType.DMA] * NBUF,
          [pltpu.SemaphoreType.DMA] * NI,
          [pltpu.SemaphoreType.DMA] * NI,
          pltpu.SemaphoreType.DMA,
      ],
  )
  def seg_kernel(h_hbm, src_hbm, dst_hbm, z_hbm, out0_hbm, out1_hbm,
                 src_v, dst_v, rows_v, acc_sh, gsems, isems, dsems, sem):
    cid = lax.axis_index("c")
    sid = lax.axis_index("s")
    wid = sid * _NC + cid
    stripe = pl.ds(sid * RPT, RPT)

    def idx_start(i, b):
      pltpu.async_copy(src_hbm.at[wid, i], src_v.at[b], isems[b])
      pltpu.async_copy(dst_hbm.at[wid, i], dst_v.at[b], dsems[b])

    def idx_wait(i, b):
      pltpu.make_async_copy(src_hbm.at[wid, i], src_v.at[b], isems[b]).wait()
      pltpu.make_async_copy(dst_hbm.at[wid, i], dst_v.at[b], dsems[b]).wait()

    def gather_start(i, rb, b):
      pltpu.async_copy(h_hbm.at[src_v.at[b]], rows_v.at[rb], gsems[rb])

    def gather_wait(i, rb, b):
      pltpu.make_async_copy(h_hbm.at[src_v.at[b]], rows_v.at[rb],
                            gsems[rb]).wait()

    for j in range(NI):
      idx_start(j, j)
    pltpu.sync_copy(z_hbm.at[stripe], acc_sh.at[stripe])
    plsc.subcore_barrier()
    for k in range(NBUF):
      idx_wait(k, k)
      gather_start(k, k, k)

    def body(g, carry):
      for bb in range(NI):
        i = g + bb
        rb = bb % NBUF
        gather_wait(i, rb, bb)
        pltpu.sync_copy(rows_v.at[rb], acc_sh.at[dst_v.at[bb]], add=True)

        @pl.when(i + NI < NCHUNK)
        def _():
          idx_start(i + NI, bb)

        @pl.when(i + NBUF < NCHUNK)
        def _():
          idx_wait(i + NBUF, (bb + NBUF) % NI)
          gather_start(i + NBUF, rb, (bb + NBUF) % NI)
      return carry

    lax.fori_loop(0, NCHUNK // NI, lambda j, c: body(j * NI, c), 0,
                  unroll=False)
    plsc.subcore_barrier()

    @pl.when(cid == 0)
    def _():
      pltpu.sync_copy(acc_sh.at[stripe], out0_hbm.at[stripe])

    @pl.when(cid == 1)
    def _():
      pltpu.sync_copy(acc_sh.at[stripe], out1_hbm.at[stripe])

  return seg_kernel(h, srcm, dstm, zeros)


_BR = 2000


def _gin_layer(a0, a1, h, W1, b1, g1, be1, W2, b2, g2, be2, relu_last,
               pool=None):
  N, D = h.shape
  Hd = W1.shape[1]
  NB = N // _BR
  inv_n = 1.0 / N
  if pool is not None:
    batchb, Wc, bc = pool
    C = Wc.shape[1]

  def body(*refs):
    if pool is not None:
      (a0_ref, a1_ref, h_ref, w1_ref, b1_ref, g1_ref, be1_ref,
       w2_ref, b2_ref, g2_ref, be2_ref, bb_ref, wc_ref, bc_ref,
       hn_ref, out_ref, z_s, z2_s, s1_s, s2_s, t1_s, t2_s,
       ps_s, pc_s) = refs
    else:
      (a0_ref, a1_ref, h_ref, w1_ref, b1_ref, g1_ref, be1_ref,
       w2_ref, b2_ref, g2_ref, be2_ref,
       hn_ref, z_s, z2_s, s1_s, s2_s, t1_s, t2_s) = refs
    i = pl.program_id(0)
    phase = i // NB
    r = i - phase * NB
    row = pl.ds(r * _BR, _BR)

    @pl.when(i == 0)
    def _():
      s1_s[...] = jnp.zeros_like(s1_s)
      s2_s[...] = jnp.zeros_like(s2_s)
      t1_s[...] = jnp.zeros_like(t1_s)
      t2_s[...] = jnp.zeros_like(t2_s)
      if pool is not None:
        ps_s[...] = jnp.zeros_like(ps_s)
        pc_s[...] = jnp.zeros_like(pc_s)

    @pl.when(phase == 0)
    def _():
      xin = a0_ref[...] + a1_ref[...] + h_ref[...]
      z = jnp.dot(xin, w1_ref[...],
                  preferred_element_type=jnp.float32) + b1_ref[...]
      z_s[row, :] = z
      s1_s[...] += jnp.broadcast_to(
          jnp.sum(z, axis=0, keepdims=True), s1_s.shape)
      s2_s[...] += jnp.broadcast_to(
          jnp.sum(z * z, axis=0, keepdims=True), s2_s.shape)

    @pl.when(phase == 1)
    def _():
      m = s1_s[0:1, :] * inv_n
      v = s2_s[0:1, :] * inv_n - m * m
      scale = lax.rsqrt(v + 1e-5) * g1_ref[...]
      zr = jnp.maximum((z_s[row, :] - m) * scale + be1_ref[...], 0.0)
      z2 = jnp.dot(zr, w2_ref[...],
                   preferred_element_type=jnp.float32) + b2_ref[...]
      z2_s[row, :] = z2
      t1_s[...] += jnp.broadcast_to(
          jnp.sum(z2, axis=0, keepdims=True), t1_s.shape)
      t2_s[...] += jnp.broadcast_to(
          jnp.sum(z2 * z2, axis=0, keepdims=True), t2_s.shape)

    @pl.when(phase == 2)
    def _():
      m = t1_s[0:1, :] * inv_n
      v = t2_s[0:1, :] * inv_n - m * m
      scale = lax.rsqrt(v + 1e-5) * g2_ref[...]
      hn = (z2_s[row, :] - m) * scale + be2_ref[...]
      if relu_last:
        hn = jnp.maximum(hn, 0.0)
      hn_ref[...] = hn
      if pool is not None:
        gid = lax.broadcasted_iota(jnp.int32, (1, _G), 1)
        onehot = (bb_ref[...] == gid).astype(jnp.float32)
        ps_s[...] += lax.dot_general(
            onehot, hn, (((0,), (0,)), ((), ())),
            preferred_element_type=jnp.float32,
            precision=lax.Precision.HIGHEST)
        pc_s[...] += lax.dot_general(
            onehot, jnp.ones_like(hn), (((0,), (0,)), ((), ())),
            preferred_element_type=jnp.float32,
            precision=lax.Precision.HIGHEST)

    if pool is not None:
      @pl.when(phase == 3)
      def _():
        hg = ps_s[...] / jnp.maximum(pc_s[...], 1.0)
        out_ref[...] = jnp.dot(
            hg, wc_ref[...], preferred_element_type=jnp.float32) + bc_ref[...]

  nsteps = 3 * NB + (1 if pool is not None else 0)
  blk = lambda i: (jnp.minimum(i, NB - 1), 0)
  cst = lambda i: (0, 0)
  out_blk = lambda i: (jnp.clip(i - 2 * NB, 0, NB - 1), 0)
  in_specs = [
      pl.BlockSpec((_BR, D), blk),
      pl.BlockSpec((_BR, D), blk),
      pl.BlockSpec((_BR, D), blk),
      pl.BlockSpec((D, Hd), cst),
      pl.BlockSpec((1, Hd), cst),
      pl.BlockSpec((1, Hd), cst),
      pl.BlockSpec((1, Hd), cst),
      pl.BlockSpec((Hd, D), cst),
      pl.BlockSpec((1, D), cst),
      pl.BlockSpec((1, D), cst),
      pl.BlockSpec((1, D), cst),
  ]
  scratch = [
      pltpu.VMEM((N, Hd), jnp.float32),
      pltpu.VMEM((N, D), jnp.float32),
      pltpu.VMEM((8, Hd), jnp.float32),
      pltpu.VMEM((8, Hd), jnp.float32),
      pltpu.VMEM((8, D), jnp.float32),
      pltpu.VMEM((8, D), jnp.float32),
  ]
  out_specs = [pl.BlockSpec((_BR, D), out_blk)]
  out_shape = [jax.ShapeDtypeStruct((N, D), jnp.float32)]
  args = [a0, a1, h, W1, b1, g1, be1, W2, b2, g2, be2]
  if pool is not None:
    in_specs += [
        pl.BlockSpec((_BR, _G), out_blk),
        pl.BlockSpec((D, C), cst),
        pl.BlockSpec((1, C), cst),
    ]
    args += [batchb, Wc, bc]
    out_specs.append(pl.BlockSpec((_G, C), cst))
    out_shape.append(jax.ShapeDtypeStruct((_G, C), jnp.float32))
    scratch += [
        pltpu.VMEM((_G, D), jnp.float32),
        pltpu.VMEM((_G, D), jnp.float32),
    ]

  res = pl.pallas_call(
      body,
      grid=(nsteps,),
      in_specs=in_specs,
      out_specs=out_specs,
      out_shape=out_shape,
      scratch_shapes=scratch,
  )(*args)
  return res


def kernel(x, edge_index, edge_attr, batch, W1, b1, g1, be1,
           W2, b2, g2, be2, Wc, bc):
  del edge_attr
  N, D = x.shape
  L = W1.shape[0]
  E = edge_index.shape[1]
  NW = _NC * _NS
  CH = 40
  srcm = edge_index[0].reshape(NW, (E // NW) // CH, CH)
  dstm = edge_index[1].reshape(NW, (E // NW) // CH, CH)
  npad = ((N + _NS * 8 - 1) // (_NS * 8)) * (_NS * 8)
  zeros = jnp.zeros((npad, D), jnp.float32)
  batchb = jnp.broadcast_to(batch[:, None], (N, _G))

  h = x
  for l in range(L):
    a0, a1 = _segment_sum_sc(h, srcm, dstm, zeros)
    last = l == L - 1
    res = _gin_layer(
        a0, a1, h, W1[l], b1[l].reshape(1, -1), g1[l].reshape(1, -1),
        be1[l].reshape(1, -1), W2[l], b2[l].reshape(1, -1),
        g2[l].reshape(1, -1), be2[l].reshape(1, -1), relu_last=not last,
        pool=(batchb, Wc, bc.reshape(1, -1)) if last else None)
    h = res[0]

  return res[1]


# speedup vs baseline: 1.0704x; 1.0259x over previous
import functools

import jax
import jax.numpy as jnp
from jax import lax
from jax.experimental import pallas as pl
from jax.experimental.pallas import tpu as pltpu
from jax.experimental.pallas import tpu_sc as plsc

_NC = 2
_NS = 16
_G = 128


def _segment_sum_sc(h, srcm, dstm, zeros):
  N, D = h.shape
  NW, NCHUNK, CH = srcm.shape
  NBUF = 5
  NI = 2 * NBUF
  NPAD = zeros.shape[0]
  RPT = NPAD // _NS

  mesh = plsc.VectorSubcoreMesh(core_axis_name="c", subcore_axis_name="s")

  @functools.partial(
      pl.kernel,
      out_type=(jax.ShapeDtypeStruct((NPAD, D), jnp.float32),
                jax.ShapeDtypeStruct((NPAD, D), jnp.float32)),
      mesh=mesh,
      scratch_types=[
          pltpu.VMEM((NI, CH), jnp.int32),
          pltpu.VMEM((NI, CH), jnp.int32),
          pltpu.VMEM((NBUF, CH, D), jnp.float32),
          pltpu.VMEM_SHARED((NPAD, D), jnp.float32),
          [pltpu.SemaphoreType.DMA] * NBUF,
          [pltpu.SemaphoreType.DMA] * NI,
          [pltpu.SemaphoreType.DMA] * NI,
          pltpu.SemaphoreType.DMA,
      ],
  )
  def seg_kernel(h_hbm, src_hbm, dst_hbm, z_hbm, out0_hbm, out1_hbm,
                 src_v, dst_v, rows_v, acc_sh, gsems, isems, dsems, sem):
    cid = lax.axis_index("c")
    sid = lax.axis_index("s")
    wid = sid * _NC + cid
    stripe = pl.ds(sid * RPT, RPT)

    def idx_start(i, b):
      pltpu.async_copy(src_hbm.at[wid, i], src_v.at[b], isems[b])
      pltpu.async_copy(dst_hbm.at[wid, i], dst_v.at[b], dsems[b])

    def idx_wait(i, b):
      pltpu.make_async_copy(src_hbm.at[wid, i], src_v.at[b], isems[b]).wait()
      pltpu.make_async_copy(dst_hbm.at[wid, i], dst_v.at[b], dsems[b]).wait()

    def gather_start(i, rb, b):
      pltpu.async_copy(h_hbm.at[src_v.at[b]], rows_v.at[rb], gsems[rb])

    def gather_wait(i, rb, b):
      pltpu.make_async_copy(h_hbm.at[src_v.at[b]], rows_v.at[rb],
                            gsems[rb]).wait()

    for j in range(NI):
      idx_start(j, j)
    for k in range(NBUF):
      idx_wait(k, k)
      gather_start(k, k, k)
    pltpu.sync_copy(z_hbm.at[stripe], acc_sh.at[stripe])
    plsc.subcore_barrier()

    def body(g, carry):
      for bb in range(NI):
        i = g + bb
        rb = bb % NBUF
        gather_wait(i, rb, bb)
        pltpu.sync_copy(rows_v.at[rb], acc_sh.at[dst_v.at[bb]], add=True)

        @pl.when(i + NI < NCHUNK)
        def _():
          idx_start(i + NI, bb)

        @pl.when(i + NBUF < NCHUNK)
        def _():
          idx_wait(i + NBUF, (bb + NBUF) % NI)
          gather_start(i + NBUF, rb, (bb + NBUF) % NI)
      return carry

    lax.fori_loop(0, NCHUNK // NI, lambda j, c: body(j * NI, c), 0,
                  unroll=False)
    plsc.subcore_barrier()

    @pl.when(cid == 0)
    def _():
      pltpu.sync_copy(acc_sh.at[stripe], out0_hbm.at[stripe])

    @pl.when(cid == 1)
    def _():
      pltpu.sync_copy(acc_sh.at[stripe], out1_hbm.at[stripe])

  return seg_kernel(h, srcm, dstm, zeros)


_BR = 5000


def _gin_layer(a0, a1, h, W1, b1, g1, be1, W2, b2, g2, be2, relu_last,
               pool=None):
  N, D = h.shape
  Hd = W1.shape[1]
  NB = N // _BR
  inv_n = 1.0 / N
  if pool is not None:
    batchb, Wc, bc = pool
    C = Wc.shape[1]

  def body(*refs):
    if pool is not None:
      (a0_ref, a1_ref, h_ref, w1_ref, b1_ref, g1_ref, be1_ref,
       w2_ref, b2_ref, g2_ref, be2_ref, bb_ref, wc_ref, bc_ref,
       hn_ref, out_ref, z_s, z2_s, s1_s, s2_s, t1_s, t2_s,
       ps_s, pc_s) = refs
    else:
      (a0_ref, a1_ref, h_ref, w1_ref, b1_ref, g1_ref, be1_ref,
       w2_ref, b2_ref, g2_ref, be2_ref,
       hn_ref, z_s, z2_s, s1_s, s2_s, t1_s, t2_s) = refs
    i = pl.program_id(0)
    phase = i // NB
    r = i - phase * NB
    row = pl.ds(r * _BR, _BR)

    @pl.when(i == 0)
    def _():
      s1_s[...] = jnp.zeros_like(s1_s)
      s2_s[...] = jnp.zeros_like(s2_s)
      t1_s[...] = jnp.zeros_like(t1_s)
      t2_s[...] = jnp.zeros_like(t2_s)
      if pool is not None:
        ps_s[...] = jnp.zeros_like(ps_s)
        pc_s[...] = jnp.zeros_like(pc_s)

    @pl.when(phase == 0)
    def _():
      xin = a0_ref[...] + a1_ref[...] + h_ref[...]
      z = jnp.dot(xin, w1_ref[...],
                  preferred_element_type=jnp.float32) + b1_ref[...]
      z_s[row, :] = z
      s1_s[...] += jnp.broadcast_to(
          jnp.sum(z, axis=0, keepdims=True), s1_s.shape)
      s2_s[...] += jnp.broadcast_to(
          jnp.sum(z * z, axis=0, keepdims=True), s2_s.shape)

    @pl.when(phase == 1)
    def _():
      m = s1_s[0:1, :] * inv_n
      v = s2_s[0:1, :] * inv_n - m * m
      scale = lax.rsqrt(v + 1e-5) * g1_ref[...]
      zr = jnp.maximum((z_s[row, :] - m) * scale + be1_ref[...], 0.0)
      z2 = jnp.dot(zr, w2_ref[...],
                   preferred_element_type=jnp.float32) + b2_ref[...]
      z2_s[row, :] = z2
      t1_s[...] += jnp.broadcast_to(
          jnp.sum(z2, axis=0, keepdims=True), t1_s.shape)
      t2_s[...] += jnp.broadcast_to(
          jnp.sum(z2 * z2, axis=0, keepdims=True), t2_s.shape)

    @pl.when(phase == 2)
    def _():
      m = t1_s[0:1, :] * inv_n
      v = t2_s[0:1, :] * inv_n - m * m
      scale = lax.rsqrt(v + 1e-5) * g2_ref[...]
      hn = (z2_s[row, :] - m) * scale + be2_ref[...]
      if relu_last:
        hn = jnp.maximum(hn, 0.0)
      hn_ref[...] = hn
      if pool is not None:
        gid = lax.broadcasted_iota(jnp.int32, (1, _G), 1)
        onehot = (bb_ref[...] == gid).astype(jnp.float32)
        ps_s[...] += lax.dot_general(
            onehot, hn, (((0,), (0,)), ((), ())),
            preferred_element_type=jnp.float32,
            precision=lax.Precision.HIGHEST)
        pc_s[...] += lax.dot_general(
            onehot, jnp.ones_like(hn), (((0,), (0,)), ((), ())),
            preferred_element_type=jnp.float32,
            precision=lax.Precision.HIGHEST)

    if pool is not None:
      @pl.when(phase == 3)
      def _():
        hg = ps_s[...] / jnp.maximum(pc_s[...], 1.0)
        out_ref[...] = jnp.dot(
            hg, wc_ref[...], preferred_element_type=jnp.float32) + bc_ref[...]

  nsteps = 3 * NB + (1 if pool is not None else 0)
  blk = lambda i: (jnp.minimum(i, NB - 1), 0)
  cst = lambda i: (0, 0)
  out_blk = lambda i: (jnp.clip(i - 2 * NB, 0, NB - 1), 0)
  in_specs = [
      pl.BlockSpec((_BR, D), blk),
      pl.BlockSpec((_BR, D), blk),
      pl.BlockSpec((_BR, D), blk),
      pl.BlockSpec((D, Hd), cst),
      pl.BlockSpec((1, Hd), cst),
      pl.BlockSpec((1, Hd), cst),
      pl.BlockSpec((1, Hd), cst),
      pl.BlockSpec((Hd, D), cst),
      pl.BlockSpec((1, D), cst),
      pl.BlockSpec((1, D), cst),
      pl.BlockSpec((1, D), cst),
  ]
  scratch = [
      pltpu.VMEM((N, Hd), jnp.float32),
      pltpu.VMEM((N, D), jnp.float32),
      pltpu.VMEM((8, Hd), jnp.float32),
      pltpu.VMEM((8, Hd), jnp.float32),
      pltpu.VMEM((8, D), jnp.float32),
      pltpu.VMEM((8, D), jnp.float32),
  ]
  out_specs = [pl.BlockSpec((_BR, D), out_blk)]
  out_shape = [jax.ShapeDtypeStruct((N, D), jnp.float32)]
  args = [a0, a1, h, W1, b1, g1, be1, W2, b2, g2, be2]
  if pool is not None:
    in_specs += [
        pl.BlockSpec((_BR, _G), out_blk),
        pl.BlockSpec((D, C), cst),
        pl.BlockSpec((1, C), cst),
    ]
    args += [batchb, Wc, bc]
    out_specs.append(pl.BlockSpec((_G, C), cst))
    out_shape.append(jax.ShapeDtypeStruct((_G, C), jnp.float32))
    scratch += [
        pltpu.VMEM((_G, D), jnp.float32),
        pltpu.VMEM((_G, D), jnp.float32),
    ]

  res = pl.pallas_call(
      body,
      grid=(nsteps,),
      in_specs=in_specs,
      out_specs=out_specs,
      out_shape=out_shape,
      scratch_shapes=scratch,
  )(*args)
  return res


def kernel(x, edge_index, edge_attr, batch, W1, b1, g1, be1,
           W2, b2, g2, be2, Wc, bc):
  del edge_attr
  N, D = x.shape
  L = W1.shape[0]
  E = edge_index.shape[1]
  NW = _NC * _NS
  CH = 40
  srcm = edge_index[0].reshape(NW, (E // NW) // CH, CH)
  dstm = edge_index[1].reshape(NW, (E // NW) // CH, CH)
  npad = ((N + _NS * 8 - 1) // (_NS * 8)) * (_NS * 8)
  zeros = jnp.zeros((npad, D), jnp.float32)
  batchb = jnp.broadcast_to(batch[:, None], (N, _G))

  h = x
  for l in range(L):
    a0, a1 = _segment_sum_sc(h, srcm, dstm, zeros)
    last = l == L - 1
    res = _gin_layer(
        a0, a1, h, W1[l], b1[l].reshape(1, -1), g1[l].reshape(1, -1),
        be1[l].reshape(1, -1), W2[l], b2[l].reshape(1, -1),
        g2[l].reshape(1, -1), be2[l].reshape(1, -1), relu_last=not last,
        pool=(batchb, Wc, bc.reshape(1, -1)) if last else None)
    h = res[0]

  return res[1]


# speedup vs baseline: 1.0833x; 1.0121x over previous
import functools

import jax
import jax.numpy as jnp
from jax import lax
from jax.experimental import pallas as pl
from jax.experimental.pallas import tpu as pltpu
from jax.experimental.pallas import tpu_sc as plsc

_NC = 2
_NS = 16
_G = 128


def _segment_sum_sc(h, srcm, dstm, zeros):
  N, D = h.shape
  NW, NCHUNK, CH = srcm.shape
  NBUF = 5
  NI = 2 * NBUF
  NPAD = zeros.shape[0]
  RPT = NPAD // _NS

  mesh = plsc.VectorSubcoreMesh(core_axis_name="c", subcore_axis_name="s")

  @functools.partial(
      pl.kernel,
      out_type=(jax.ShapeDtypeStruct((NPAD, D), jnp.float32),
                jax.ShapeDtypeStruct((NPAD, D), jnp.float32)),
      mesh=mesh,
      scratch_types=[
          pltpu.VMEM((NI, CH), jnp.int32),
          pltpu.VMEM((NI, CH), jnp.int32),
          pltpu.VMEM((NBUF, CH, D), jnp.float32),
          pltpu.VMEM_SHARED((NPAD, D), jnp.float32),
          [pltpu.SemaphoreType.DMA] * NBUF,
          [pltpu.SemaphoreType.DMA] * NI,
          [pltpu.SemaphoreType.DMA] * NI,
          pltpu.SemaphoreType.DMA,
      ],
  )
  def seg_kernel(h_hbm, src_hbm, dst_hbm, z_hbm, out0_hbm, out1_hbm,
                 src_v, dst_v, rows_v, acc_sh, gsems, isems, dsems, sem):
    cid = lax.axis_index("c")
    sid = lax.axis_index("s")
    wid = sid * _NC + cid
    stripe = pl.ds(sid * RPT, RPT)

    def idx_start(i, b):
      pltpu.async_copy(src_hbm.at[wid, i], src_v.at[b], isems[b])
      pltpu.async_copy(dst_hbm.at[wid, i], dst_v.at[b], dsems[b])

    def idx_wait(i, b):
      pltpu.make_async_copy(src_hbm.at[wid, i], src_v.at[b], isems[b]).wait()
      pltpu.make_async_copy(dst_hbm.at[wid, i], dst_v.at[b], dsems[b]).wait()

    def gather_start(i, rb, b):
      pltpu.async_copy(h_hbm.at[src_v.at[b]], rows_v.at[rb], gsems[rb])

    def gather_wait(i, rb, b):
      pltpu.make_async_copy(h_hbm.at[src_v.at[b]], rows_v.at[rb],
                            gsems[rb]).wait()

    for j in range(NI):
      idx_start(j, j)
    for k in range(NBUF):
      idx_wait(k, k)
      gather_start(k, k, k)

    NT = N - (_NS - 1) * RPT

    @pl.when(cid == 0)
    def _():
      @pl.when(sid < _NS - 1)
      def _():
        pltpu.sync_copy(h_hbm.at[stripe], acc_sh.at[stripe])

      @pl.when(sid == _NS - 1)
      def _():
        tail = pl.ds((_NS - 1) * RPT, NT)
        pltpu.sync_copy(h_hbm.at[tail], acc_sh.at[tail])
        pad = pl.ds(N, NPAD - N)
        pltpu.sync_copy(z_hbm.at[pad], acc_sh.at[pad])

    @pl.when(cid == 1)
    def _():
      pltpu.sync_copy(z_hbm.at[stripe], acc_sh.at[stripe])
    plsc.subcore_barrier()

    def body(g, carry):
      for bb in range(NI):
        i = g + bb
        rb = bb % NBUF
        gather_wait(i, rb, bb)
        pltpu.sync_copy(rows_v.at[rb], acc_sh.at[dst_v.at[bb]], add=True)

        @pl.when(i + NI < NCHUNK)
        def _():
          idx_start(i + NI, bb)

        @pl.when(i + NBUF < NCHUNK)
        def _():
          idx_wait(i + NBUF, (bb + NBUF) % NI)
          gather_start(i + NBUF, rb, (bb + NBUF) % NI)
      return carry

    lax.fori_loop(0, NCHUNK // NI, lambda j, c: body(j * NI, c), 0,
                  unroll=False)
    plsc.subcore_barrier()

    @pl.when(cid == 0)
    def _():
      pltpu.sync_copy(acc_sh.at[stripe], out0_hbm.at[stripe])

    @pl.when(cid == 1)
    def _():
      pltpu.sync_copy(acc_sh.at[stripe], out1_hbm.at[stripe])

  return seg_kernel(h, srcm, dstm, zeros)


_BR = 5000


def _gin_layer(n, a0, a1, W1, b1, g1, be1, W2, b2, g2, be2, relu_last,
               pool=None):
  N = n
  D = W1.shape[0]
  Hd = W1.shape[1]
  NB = N // _BR
  inv_n = 1.0 / N
  if pool is not None:
    batchb, Wc, bc = pool
    C = Wc.shape[1]

  def body(*refs):
    if pool is not None:
      (a0_ref, a1_ref, w1_ref, b1_ref, g1_ref, be1_ref,
       w2_ref, b2_ref, g2_ref, be2_ref, bb_ref, wc_ref, bc_ref,
       hn_ref, out_ref, z_s, z2_s, s1_s, s2_s, t1_s, t2_s,
       ps_s, pc_s) = refs
    else:
      (a0_ref, a1_ref, w1_ref, b1_ref, g1_ref, be1_ref,
       w2_ref, b2_ref, g2_ref, be2_ref,
       hn_ref, z_s, z2_s, s1_s, s2_s, t1_s, t2_s) = refs
    i = pl.program_id(0)
    phase = i // NB
    r = i - phase * NB
    row = pl.ds(r * _BR, _BR)

    @pl.when(i == 0)
    def _():
      s1_s[...] = jnp.zeros_like(s1_s)
      s2_s[...] = jnp.zeros_like(s2_s)
      t1_s[...] = jnp.zeros_like(t1_s)
      t2_s[...] = jnp.zeros_like(t2_s)
      if pool is not None:
        ps_s[...] = jnp.zeros_like(ps_s)
        pc_s[...] = jnp.zeros_like(pc_s)

    @pl.when(phase == 0)
    def _():
      xin = a0_ref[...] + a1_ref[...]
      z = jnp.dot(xin, w1_ref[...],
                  preferred_element_type=jnp.float32) + b1_ref[...]
      z_s[row, :] = z
      s1_s[...] += jnp.broadcast_to(
          jnp.sum(z, axis=0, keepdims=True), s1_s.shape)
      s2_s[...] += jnp.broadcast_to(
          jnp.sum(z * z, axis=0, keepdims=True), s2_s.shape)

    @pl.when(phase == 1)
    def _():
      m = s1_s[0:1, :] * inv_n
      v = s2_s[0:1, :] * inv_n - m * m
      scale = lax.rsqrt(v + 1e-5) * g1_ref[...]
      zr = jnp.maximum((z_s[row, :] - m) * scale + be1_ref[...], 0.0)
      z2 = jnp.dot(zr, w2_ref[...],
                   preferred_element_type=jnp.float32) + b2_ref[...]
      z2_s[row, :] = z2
      t1_s[...] += jnp.broadcast_to(
          jnp.sum(z2, axis=0, keepdims=True), t1_s.shape)
      t2_s[...] += jnp.broadcast_to(
          jnp.sum(z2 * z2, axis=0, keepdims=True), t2_s.shape)

    @pl.when(phase == 2)
    def _():
      m = t1_s[0:1, :] * inv_n
      v = t2_s[0:1, :] * inv_n - m * m
      scale = lax.rsqrt(v + 1e-5) * g2_ref[...]
      hn = (z2_s[row, :] - m) * scale + be2_ref[...]
      if relu_last:
        hn = jnp.maximum(hn, 0.0)
      hn_ref[...] = hn
      if pool is not None:
        gid = lax.broadcasted_iota(jnp.int32, (1, _G), 1)
        onehot = (bb_ref[...] == gid).astype(jnp.float32)
        ps_s[...] += lax.dot_general(
            onehot, hn, (((0,), (0,)), ((), ())),
            preferred_element_type=jnp.float32,
            precision=lax.Precision.HIGHEST)
        pc_s[...] += lax.dot_general(
            onehot, jnp.ones_like(hn), (((0,), (0,)), ((), ())),
            preferred_element_type=jnp.float32,
            precision=lax.Precision.HIGHEST)

    if pool is not None:
      @pl.when(phase == 3)
      def _():
        hg = ps_s[...] / jnp.maximum(pc_s[...], 1.0)
        out_ref[...] = jnp.dot(
            hg, wc_ref[...], preferred_element_type=jnp.float32) + bc_ref[...]

  nsteps = 3 * NB + (1 if pool is not None else 0)
  blk = lambda i: (jnp.minimum(i, NB - 1), 0)
  cst = lambda i: (0, 0)
  out_blk = lambda i: (jnp.clip(i - 2 * NB, 0, NB - 1), 0)
  in_specs = [
      pl.BlockSpec((_BR, D), blk),
      pl.BlockSpec((_BR, D), blk),
      pl.BlockSpec((D, Hd), cst),
      pl.BlockSpec((1, Hd), cst),
      pl.BlockSpec((1, Hd), cst),
      pl.BlockSpec((1, Hd), cst),
      pl.BlockSpec((Hd, D), cst),
      pl.BlockSpec((1, D), cst),
      pl.BlockSpec((1, D), cst),
      pl.BlockSpec((1, D), cst),
  ]
  scratch = [
      pltpu.VMEM((N, Hd), jnp.float32),
      pltpu.VMEM((N, D), jnp.float32),
      pltpu.VMEM((8, Hd), jnp.float32),
      pltpu.VMEM((8, Hd), jnp.float32),
      pltpu.VMEM((8, D), jnp.float32),
      pltpu.VMEM((8, D), jnp.float32),
  ]
  out_specs = [pl.BlockSpec((_BR, D), out_blk)]
  out_shape = [jax.ShapeDtypeStruct((N, D), jnp.float32)]
  args = [a0, a1, W1, b1, g1, be1, W2, b2, g2, be2]
  if pool is not None:
    in_specs += [
        pl.BlockSpec((_BR, _G), out_blk),
        pl.BlockSpec((D, C), cst),
        pl.BlockSpec((1, C), cst),
    ]
    args += [batchb, Wc, bc]
    out_specs.append(pl.BlockSpec((_G, C), cst))
    out_shape.append(jax.ShapeDtypeStruct((_G, C), jnp.float32))
    scratch += [
        pltpu.VMEM((_G, D), jnp.float32),
        pltpu.VMEM((_G, D), jnp.float32),
    ]

  res = pl.pallas_call(
      body,
      grid=(nsteps,),
      in_specs=in_specs,
      out_specs=out_specs,
      out_shape=out_shape,
      scratch_shapes=scratch,
  )(*args)
  return res


def kernel(x, edge_index, edge_attr, batch, W1, b1, g1, be1,
           W2, b2, g2, be2, Wc, bc):
  del edge_attr
  N, D = x.shape
  L = W1.shape[0]
  E = edge_index.shape[1]
  NW = _NC * _NS
  CH = 40
  srcm = edge_index[0].reshape(NW, (E // NW) // CH, CH)
  dstm = edge_index[1].reshape(NW, (E // NW) // CH, CH)
  npad = ((N + _NS * 8 - 1) // (_NS * 8)) * (_NS * 8)
  zeros = jnp.zeros((npad, D), jnp.float32)
  batchb = jnp.broadcast_to(batch[:, None], (N, _G))

  h = x
  for l in range(L):
    a0, a1 = _segment_sum_sc(h, srcm, dstm, zeros)
    last = l == L - 1
    res = _gin_layer(
        N, a0, a1, W1[l], b1[l].reshape(1, -1), g1[l].reshape(1, -1),
        be1[l].reshape(1, -1), W2[l], b2[l].reshape(1, -1),
        g2[l].reshape(1, -1), be2[l].reshape(1, -1), relu_last=not last,
        pool=(batchb, Wc, bc.reshape(1, -1)) if last else None)
    h = res[0]

  return res[1]
